# baseline (speedup 1.0000x reference)
import functools

import jax
import jax.numpy as jnp
from jax import lax
from jax.experimental import pallas as pl
from jax.experimental.pallas import tpu as pltpu
import jax.experimental.pallas.tpu_sc as plsc

N = 10000
E = 160000
IN_DIM = 256
HID_DIM = 256
LAT_DIM = 128
OUT_DIM = 256

NC = 2
NT = 16
HALF = N // NC
ACC_ROWS = HALF + 8
TRASH = HALF
EPT = E // NT
CH = 100
NCHM = 102
LIST_PAD = NCHM * CH
IVEC = 16
INIT_TILES = 8
INIT_ROWS = HALF // INIT_TILES
INIT_CHUNKS = INIT_ROWS // CH
INIT_TAIL = INIT_ROWS - INIT_CHUNKS * CH

_MESH = plsc.VectorSubcoreMesh(core_axis_name="c", subcore_axis_name="s")
_SC_PARAMS = pltpu.CompilerParams(use_tc_tiling_on_sc=False,
                                  needs_layout_passes=False)


def _build_part(interpret=False):
    return functools.partial(
        pl.kernel,
        out_type=[
            jax.ShapeDtypeStruct((NC, NT, LIST_PAD), jnp.int32),
            jax.ShapeDtypeStruct((NC, NT, NCHM, CH), jnp.int32),
            jax.ShapeDtypeStruct((NC, NT, 16), jnp.int32),
            jax.ShapeDtypeStruct((N, 16), jnp.float32),
        ],
        mesh=_MESH,
        scratch_types=[
            pltpu.VMEM((EPT,), jnp.int32),
            pltpu.VMEM((EPT,), jnp.int32),
            pltpu.VMEM((LIST_PAD,), jnp.int32),
            pltpu.VMEM((NCHM, CH), jnp.int32),
            pltpu.VMEM((16,), jnp.int32),
            pltpu.VMEM((CH, 16), jnp.float32),
            pltpu.VMEM((INIT_ROWS, 16), jnp.float32),
            pltpu.VMEM_SHARED((ACC_ROWS, 16), jnp.float32),
            pltpu.SemaphoreType.DMA,
        ],
        compiler_params=_SC_PARAMS,
        interpret=interpret,
    )(_part_body)


def _part_body(src_hbm, dst_hbm, ones_hbm,
               lsrc_hbm, ldst_hbm, cnt_hbm, deg_hbm,
               sbuf, dbuf, lsrc, ldst, cntb, ones_v, stage, dacc, sem):
    c = lax.axis_index("c")
    s = lax.axis_index("s")
    lo = c * HALF
    pltpu.sync_copy(src_hbm.at[s], sbuf)
    pltpu.sync_copy(dst_hbm.at[s], dbuf)
    pltpu.sync_copy(ones_hbm.at[pl.ds(0, CH)], ones_v)

    @pl.when(s < INIT_TILES)
    def _():
        pltpu.sync_copy(ones_hbm, stage)
        pltpu.sync_copy(stage, dacc.at[pl.ds(s * INIT_ROWS, INIT_ROWS)])

    def filt(i, cnt):
        d = dbuf[pl.ds(i * IVEC, IVEC)]
        sr = sbuf[pl.ds(i * IVEC, IVEC)]
        m = (d >= lo) & (d < lo + HALF)
        mi = jnp.where(m, jnp.full((IVEC,), 1, jnp.int32),
                       jnp.full((IVEC,), 0, jnp.int32))
        incl = plsc.cumsum(mi)
        pos = cnt + incl - 1
        plsc.store_scatter(lsrc, [pos], sr, mask=m)
        plsc.store_scatter(ldst, [pos // CH, pos % CH], d - lo, mask=m)
        return cnt + incl[IVEC - 1]

    cnt = lax.fori_loop(0, EPT // IVEC, filt, jnp.int32(0))

    lanes = lax.iota(jnp.int32, IVEC)
    for k in range(CH // IVEC + 1):
        pos = cnt + k * IVEC + lanes
        plsc.store_scatter(lsrc, [pos], jnp.zeros((IVEC,), jnp.int32))
        plsc.store_scatter(ldst, [pos // CH, pos % CH],
                           jnp.full((IVEC,), TRASH, jnp.int32))

    nch = (cnt + (CH - 1)) // CH
    cntb[...] = jnp.broadcast_to(nch, (IVEC,))
    pltpu.sync_copy(cntb, cnt_hbm.at[c, s])
    pltpu.sync_copy(lsrc, lsrc_hbm.at[c, s])
    pltpu.sync_copy(ldst, ldst_hbm.at[c, s])
    plsc.subcore_barrier()

    def dchunk(j, carry):
        pltpu.sync_copy(ones_v, dacc.at[ldst.at[j]], add=True)
        return carry

    lax.fori_loop(0, nch, dchunk, 0)
    plsc.subcore_barrier()

    @pl.when(s < INIT_TILES)
    def _():
        pltpu.sync_copy(dacc.at[pl.ds(s * INIT_ROWS, INIT_ROWS)], stage)
        pltpu.sync_copy(stage,
                        deg_hbm.at[pl.ds(c * HALF + s * INIT_ROWS, INIT_ROWS)])


_part_kernel = _build_part()


def _make_prop(dh, interpret=False):
    @functools.partial(
        pl.kernel,
        out_type=jax.ShapeDtypeStruct((N, dh), jnp.float32),
        mesh=_MESH,
        scratch_types=[
            pltpu.VMEM((NCHM, CH), jnp.int32),
            pltpu.VMEM((NCHM, CH), jnp.int32),
            pltpu.VMEM((CH, dh), jnp.float32),
            pltpu.VMEM((16,), jnp.int32),
            pltpu.VMEM_SHARED((ACC_ROWS, dh), jnp.float32),
            pltpu.SemaphoreType.DMA,
        ],
        compiler_params=_SC_PARAMS,
        interpret=interpret,
    )
    def prop(y_hbm, lsrc_hbm, ldst_hbm, cnt_hbm, out_hbm,
             lsrc, ldst, rows, cntb, acc, sem):
        c = lax.axis_index("c")
        s = lax.axis_index("s")
        pltpu.sync_copy(cnt_hbm.at[c, s], cntb)
        pltpu.sync_copy(lsrc_hbm.at[c, s], lsrc)
        pltpu.sync_copy(ldst_hbm.at[c, s], ldst)

        @pl.when(s < INIT_TILES)
        def _():
            def init(k, carry):
                pltpu.sync_copy(
                    y_hbm.at[pl.ds(c * HALF + s * INIT_ROWS + k * CH, CH)], rows)
                pltpu.sync_copy(rows, acc.at[pl.ds(s * INIT_ROWS + k * CH, CH)])
                return carry

            lax.fori_loop(0, INIT_CHUNKS, init, 0)
            pltpu.sync_copy(
                y_hbm.at[pl.ds(c * HALF + s * INIT_ROWS + INIT_CHUNKS * CH,
                               INIT_TAIL)],
                rows.at[pl.ds(0, INIT_TAIL)])
            pltpu.sync_copy(rows.at[pl.ds(0, INIT_TAIL)],
                            acc.at[pl.ds(s * INIT_ROWS + INIT_CHUNKS * CH,
                                         INIT_TAIL)])

        plsc.subcore_barrier()
        nch = cntb[...][0]

        def chunk(j, carry):
            pltpu.async_copy(y_hbm.at[lsrc.at[j]], rows, sem).wait()
            pltpu.sync_copy(rows, acc.at[ldst.at[j]], add=True)
            return carry

        lax.fori_loop(0, nch, chunk, 0)
        plsc.subcore_barrier()

        @pl.when(s < INIT_TILES)
        def _():
            def drain(k, carry):
                pltpu.sync_copy(acc.at[pl.ds(s * INIT_ROWS + k * CH, CH)], rows)
                pltpu.sync_copy(
                    rows,
                    out_hbm.at[pl.ds(c * HALF + s * INIT_ROWS + k * CH, CH)])
                return carry

            lax.fori_loop(0, INIT_CHUNKS, drain, 0)
            pltpu.sync_copy(acc.at[pl.ds(s * INIT_ROWS + INIT_CHUNKS * CH,
                                         INIT_TAIL)],
                            rows.at[pl.ds(0, INIT_TAIL)])
            pltpu.sync_copy(
                rows.at[pl.ds(0, INIT_TAIL)],
                out_hbm.at[pl.ds(c * HALF + s * INIT_ROWS + INIT_CHUNKS * CH,
                                 INIT_TAIL)])

    return prop


_prop256 = _make_prop(256)
_prop128 = _make_prop(128)


_R = 1000
_GRID = N // _R


def _tc1_body(x_ref, w_ref, b_ref, deg_ref, y_ref, dis_ref):
    dis = lax.rsqrt(deg_ref[...][:, :1])
    h = jnp.dot(x_ref[...], w_ref[...], preferred_element_type=jnp.float32)
    y_ref[...] = (h + b_ref[...]) * dis
    dis_ref[...] = dis


def _tc1(x, w1, b1, deg):
    return pl.pallas_call(
        _tc1_body,
        grid=(_GRID,),
        in_specs=[
            pl.BlockSpec((_R, IN_DIM), lambda i: (i, 0)),
            pl.BlockSpec((IN_DIM, HID_DIM), lambda i: (0, 0)),
            pl.BlockSpec((1, HID_DIM), lambda i: (0, 0)),
            pl.BlockSpec((_R, 16), lambda i: (i, 0)),
        ],
        out_specs=[
            pl.BlockSpec((_R, HID_DIM), lambda i: (i, 0)),
            pl.BlockSpec((_R, 1), lambda i: (i, 0)),
        ],
        out_shape=[
            jax.ShapeDtypeStruct((N, HID_DIM), jnp.float32),
            jax.ShapeDtypeStruct((N, 1), jnp.float32),
        ],
    )(x, w1, b1, deg)


def _tc2_body(a_ref, w_ref, b_ref, dis_ref, y_ref):
    dis = dis_ref[...]
    h = jnp.maximum(a_ref[...] * dis, 0.0)
    y = jnp.dot(h, w_ref[...], preferred_element_type=jnp.float32)
    y_ref[...] = (y + b_ref[...]) * dis


def _tc2(acc1, w2, b2, dis):
    return pl.pallas_call(
        _tc2_body,
        grid=(_GRID,),
        in_specs=[
            pl.BlockSpec((_R, HID_DIM), lambda i: (i, 0)),
            pl.BlockSpec((HID_DIM, LAT_DIM), lambda i: (0, 0)),
            pl.BlockSpec((1, LAT_DIM), lambda i: (0, 0)),
            pl.BlockSpec((_R, 1), lambda i: (i, 0)),
        ],
        out_specs=pl.BlockSpec((_R, LAT_DIM), lambda i: (i, 0)),
        out_shape=jax.ShapeDtypeStruct((N, LAT_DIM), jnp.float32),
    )(acc1, w2, b2, dis)


def _tc3_body(a_ref, w_ref, b_ref, dis_ref, y_ref):
    dis = dis_ref[...]
    z = a_ref[...] * dis
    y = jnp.dot(z, w_ref[...], preferred_element_type=jnp.float32)
    y_ref[...] = (y + b_ref[...]) * dis


def _tc3(acc2, w3, b3, dis):
    return pl.pallas_call(
        _tc3_body,
        grid=(_GRID,),
        in_specs=[
            pl.BlockSpec((_R, LAT_DIM), lambda i: (i, 0)),
            pl.BlockSpec((LAT_DIM, OUT_DIM), lambda i: (0, 0)),
            pl.BlockSpec((1, OUT_DIM), lambda i: (0, 0)),
            pl.BlockSpec((_R, 1), lambda i: (i, 0)),
        ],
        out_specs=pl.BlockSpec((_R, OUT_DIM), lambda i: (i, 0)),
        out_shape=jax.ShapeDtypeStruct((N, OUT_DIM), jnp.float32),
    )(acc2, w3, b3, dis)


def _tc4_body(a_ref, dis_ref, o_ref):
    o_ref[...] = a_ref[...] * dis_ref[...]


def _tc4(acc3, dis):
    return pl.pallas_call(
        _tc4_body,
        grid=(_GRID,),
        in_specs=[
            pl.BlockSpec((_R, OUT_DIM), lambda i: (i, 0)),
            pl.BlockSpec((_R, 1), lambda i: (i, 0)),
        ],
        out_specs=pl.BlockSpec((_R, OUT_DIM), lambda i: (i, 0)),
        out_shape=jax.ShapeDtypeStruct((N, OUT_DIM), jnp.float32),
    )(acc3, dis)


def kernel(x, edge_index, W1, b1, W2, b2, W3, b3):
    src = edge_index[0].astype(jnp.int32).reshape(NT, EPT)
    dst = edge_index[1].astype(jnp.int32).reshape(NT, EPT)
    ones_n = jnp.ones((INIT_ROWS, 16), jnp.float32)

    lsrc, ldst, cnts, deg = _part_kernel(src, dst, ones_n)
    lsrc = lsrc.reshape(NC, NT, NCHM, CH)

    y1, dis = _tc1(x, W1, b1.reshape(1, HID_DIM), deg)
    acc1 = _prop256(y1, lsrc, ldst, cnts)
    y2 = _tc2(acc1, W2, b2.reshape(1, LAT_DIM), dis)
    acc2 = _prop128(y2, lsrc, ldst, cnts)
    y3 = _tc3(acc2, W3, b3.reshape(1, OUT_DIM), dis)
    acc3 = _prop256(y3, lsrc, ldst, cnts)
    return _tc4(acc3, dis)


# speedup vs baseline: 1.2449x; 1.2449x over previous
import functools

import jax
import jax.numpy as jnp
from jax import lax
from jax.experimental import pallas as pl
from jax.experimental.pallas import tpu as pltpu
import jax.experimental.pallas.tpu_sc as plsc

N = 10000
E = 160000
IN_DIM = 256
HID_DIM = 256
LAT_DIM = 128
OUT_DIM = 256

NC = 2
NT = 16
ROWS_PT = N // NT
EPT = E // NT
CH = 100
NCHUNK = EPT // CH
INIT_CHUNKS = ROWS_PT // CH
INIT_TAIL = ROWS_PT - INIT_CHUNKS * CH
DEG_EPT = E // (NT * NC)
DEG_CH = 100
DEG_NCHUNK = DEG_EPT // DEG_CH

_MESH = plsc.VectorSubcoreMesh(core_axis_name="c", subcore_axis_name="s")
_SC_PARAMS = pltpu.CompilerParams(use_tc_tiling_on_sc=False,
                                  needs_layout_passes=False)


def _build_deg(interpret=False):
    return functools.partial(
        pl.kernel,
        out_type=jax.ShapeDtypeStruct((NC * N, 16), jnp.float32),
        mesh=_MESH,
        scratch_types=[
            pltpu.VMEM((DEG_NCHUNK, DEG_CH), jnp.int32),
            pltpu.VMEM((DEG_CH, 16), jnp.float32),
            pltpu.VMEM((ROWS_PT, 16), jnp.float32),
            pltpu.VMEM_SHARED((N, 16), jnp.float32),
            pltpu.SemaphoreType.DMA,
        ],
        compiler_params=_SC_PARAMS,
        interpret=interpret,
    )(_deg_body)


def _deg_body(dst_hbm, ones_hbm, out_hbm, dstbuf, ones_v, stage, acc, sem):
    c = lax.axis_index("c")
    s = lax.axis_index("s")
    w = c * NT + s
    pltpu.sync_copy(dst_hbm.at[w], dstbuf)
    pltpu.sync_copy(ones_hbm.at[pl.ds(0, DEG_CH)], ones_v)
    pltpu.sync_copy(ones_hbm, stage)
    pltpu.sync_copy(stage, acc.at[pl.ds(s * ROWS_PT, ROWS_PT)])
    plsc.subcore_barrier()

    def chunk(j, carry):
        pltpu.sync_copy(ones_v, acc.at[dstbuf.at[j]], add=True)
        return carry

    lax.fori_loop(0, DEG_NCHUNK, chunk, 0)
    plsc.subcore_barrier()
    pltpu.sync_copy(acc.at[pl.ds(s * ROWS_PT, ROWS_PT)], stage)
    pltpu.sync_copy(stage, out_hbm.at[pl.ds(c * N + s * ROWS_PT, ROWS_PT)])


_deg_kernel = _build_deg()


def _make_prop(dh, interpret=False):
    @functools.partial(
        pl.kernel,
        out_type=jax.ShapeDtypeStruct((NC * N, dh), jnp.float32),
        mesh=_MESH,
        scratch_types=[
            pltpu.VMEM((NCHUNK, CH), jnp.int32),
            pltpu.VMEM((NCHUNK, CH), jnp.int32),
            pltpu.VMEM((CH, dh), jnp.float32),
            pltpu.VMEM_SHARED((N, dh), jnp.float32),
            pltpu.SemaphoreType.DMA,
        ],
        compiler_params=_SC_PARAMS,
        interpret=interpret,
    )
    def prop(y_hbm, srcb_hbm, dstb_hbm, out_hbm, srcb, dstb, rows, acc, sem):
        c = lax.axis_index("c")
        s = lax.axis_index("s")
        pltpu.sync_copy(srcb_hbm.at[c, s], srcb)
        pltpu.sync_copy(dstb_hbm.at[s], dstb)

        def init(k, carry):
            pltpu.sync_copy(y_hbm.at[pl.ds(c * N + s * ROWS_PT + k * CH, CH)],
                            rows)
            pltpu.sync_copy(rows, acc.at[pl.ds(s * ROWS_PT + k * CH, CH)])
            return carry

        lax.fori_loop(0, INIT_CHUNKS, init, 0)
        pltpu.sync_copy(
            y_hbm.at[pl.ds(c * N + s * ROWS_PT + INIT_CHUNKS * CH, INIT_TAIL)],
            rows.at[pl.ds(0, INIT_TAIL)])
        pltpu.sync_copy(rows.at[pl.ds(0, INIT_TAIL)],
                        acc.at[pl.ds(s * ROWS_PT + INIT_CHUNKS * CH, INIT_TAIL)])
        plsc.subcore_barrier()

        def chunk(j, carry):
            pltpu.async_copy(y_hbm.at[srcb.at[j]], rows, sem).wait()
            pltpu.sync_copy(rows, acc.at[dstb.at[j]], add=True)
            return carry

        lax.fori_loop(0, NCHUNK, chunk, 0)
        plsc.subcore_barrier()

        def drain(k, carry):
            pltpu.sync_copy(acc.at[pl.ds(s * ROWS_PT + k * CH, CH)], rows)
            pltpu.sync_copy(rows,
                            out_hbm.at[pl.ds(c * N + s * ROWS_PT + k * CH, CH)])
            return carry

        lax.fori_loop(0, INIT_CHUNKS, drain, 0)
        pltpu.sync_copy(acc.at[pl.ds(s * ROWS_PT + INIT_CHUNKS * CH, INIT_TAIL)],
                        rows.at[pl.ds(0, INIT_TAIL)])
        pltpu.sync_copy(
            rows.at[pl.ds(0, INIT_TAIL)],
            out_hbm.at[pl.ds(c * N + s * ROWS_PT + INIT_CHUNKS * CH, INIT_TAIL)])

    return prop


_prop128 = _make_prop(128)
_prop64 = _make_prop(64)


_R = 1000
_GRID = N // _R


def _tc1_body(x_ref, w_ref, b_ref, d_ref, y_ref, dis_ref):
    dis = lax.rsqrt(d_ref[0][:, :1] + d_ref[1][:, :1] - 1.0)
    h = jnp.dot(x_ref[...], w_ref[...], preferred_element_type=jnp.float32)
    h = (h + b_ref[...]) * dis
    y_ref[0] = h[:, :128]
    y_ref[1] = h[:, 128:]
    dis_ref[...] = dis


def _tc1(x, w1, b1, deg):
    return pl.pallas_call(
        _tc1_body,
        grid=(_GRID,),
        in_specs=[
            pl.BlockSpec((_R, IN_DIM), lambda i: (i, 0)),
            pl.BlockSpec((IN_DIM, HID_DIM), lambda i: (0, 0)),
            pl.BlockSpec((1, HID_DIM), lambda i: (0, 0)),
            pl.BlockSpec((NC, _R, 16), lambda i: (0, i, 0)),
        ],
        out_specs=[
            pl.BlockSpec((NC, _R, 128), lambda i: (0, i, 0)),
            pl.BlockSpec((_R, 1), lambda i: (i, 0)),
        ],
        out_shape=[
            jax.ShapeDtypeStruct((NC, N, 128), jnp.float32),
            jax.ShapeDtypeStruct((N, 1), jnp.float32),
        ],
    )(x, w1, b1, deg)


def _tc2_body(a_ref, w_ref, b_ref, dis_ref, y_ref):
    dis = dis_ref[...]
    h0 = jnp.maximum(a_ref[0] * dis, 0.0)
    h1 = jnp.maximum(a_ref[1] * dis, 0.0)
    y = jnp.dot(h0, w_ref[:128], preferred_element_type=jnp.float32)
    y = y + jnp.dot(h1, w_ref[128:], preferred_element_type=jnp.float32)
    y = (y + b_ref[...]) * dis
    y_ref[0] = y[:, :64]
    y_ref[1] = y[:, 64:]


def _tc2(acc1, w2, b2, dis):
    return pl.pallas_call(
        _tc2_body,
        grid=(_GRID,),
        in_specs=[
            pl.BlockSpec((NC, _R, 128), lambda i: (0, i, 0)),
            pl.BlockSpec((HID_DIM, LAT_DIM), lambda i: (0, 0)),
            pl.BlockSpec((1, LAT_DIM), lambda i: (0, 0)),
            pl.BlockSpec((_R, 1), lambda i: (i, 0)),
        ],
        out_specs=pl.BlockSpec((NC, _R, 64), lambda i: (0, i, 0)),
        out_shape=jax.ShapeDtypeStruct((NC, N, 64), jnp.float32),
    )(acc1, w2, b2, dis)


def _tc3_body(a_ref, w_ref, b_ref, dis_ref, y_ref):
    dis = dis_ref[...]
    z0 = a_ref[0] * dis
    z1 = a_ref[1] * dis
    y = jnp.dot(z0, w_ref[:64], preferred_element_type=jnp.float32)
    y = y + jnp.dot(z1, w_ref[64:], preferred_element_type=jnp.float32)
    y = (y + b_ref[...]) * dis
    y_ref[0] = y[:, :128]
    y_ref[1] = y[:, 128:]


def _tc3(acc2, w3, b3, dis):
    return pl.pallas_call(
        _tc3_body,
        grid=(_GRID,),
        in_specs=[
            pl.BlockSpec((NC, _R, 64), lambda i: (0, i, 0)),
            pl.BlockSpec((LAT_DIM, OUT_DIM), lambda i: (0, 0)),
            pl.BlockSpec((1, OUT_DIM), lambda i: (0, 0)),
            pl.BlockSpec((_R, 1), lambda i: (i, 0)),
        ],
        out_specs=pl.BlockSpec((NC, _R, 128), lambda i: (0, i, 0)),
        out_shape=jax.ShapeDtypeStruct((NC, N, 128), jnp.float32),
    )(acc2, w3, b3, dis)


def _tc4_body(a_ref, dis_ref, o_ref):
    dis = dis_ref[...]
    o_ref[:, :128] = a_ref[0] * dis
    o_ref[:, 128:] = a_ref[1] * dis


def _tc4(acc3, dis):
    return pl.pallas_call(
        _tc4_body,
        grid=(_GRID,),
        in_specs=[
            pl.BlockSpec((NC, _R, 128), lambda i: (0, i, 0)),
            pl.BlockSpec((_R, 1), lambda i: (i, 0)),
        ],
        out_specs=pl.BlockSpec((_R, OUT_DIM), lambda i: (i, 0)),
        out_shape=jax.ShapeDtypeStruct((N, OUT_DIM), jnp.float32),
    )(acc3, dis)


def kernel(x, edge_index, W1, b1, W2, b2, W3, b3):
    src = edge_index[0].astype(jnp.int32)
    dst = edge_index[1].astype(jnp.int32)
    dstb = dst.reshape(NT, NCHUNK, CH)
    srcb = jnp.stack([src, src + N]).reshape(NC, NT, NCHUNK, CH)
    dst_deg = dst.reshape(NC * NT, DEG_NCHUNK, DEG_CH)
    ones16 = jnp.ones((ROWS_PT, 16), jnp.float32)

    deg = _deg_kernel(dst_deg, ones16).reshape(NC, N, 16)

    y1, dis = _tc1(x, W1, b1.reshape(1, HID_DIM), deg)
    acc1 = _prop128(y1.reshape(NC * N, 128), srcb, dstb)
    y2 = _tc2(acc1.reshape(NC, N, 128), W2, b2.reshape(1, LAT_DIM), dis)
    acc2 = _prop64(y2.reshape(NC * N, 64), srcb, dstb)
    y3 = _tc3(acc2.reshape(NC, N, 64), W3, b3.reshape(1, OUT_DIM), dis)
    acc3 = _prop128(y3.reshape(NC * N, 128), srcb, dstb)
    return _tc4(acc3.reshape(NC, N, 128), dis)


# speedup vs baseline: 1.3425x; 1.0784x over previous
import functools

import jax
import jax.numpy as jnp
from jax import lax
from jax.experimental import pallas as pl
from jax.experimental.pallas import tpu as pltpu
import jax.experimental.pallas.tpu_sc as plsc

N = 10000
E = 160000
IN_DIM = 256
HID_DIM = 256
LAT_DIM = 128
OUT_DIM = 256

NC = 2
NT = 16
ROWS_PT = N // NT
EPT = E // NT
CH = 125
NCHUNK = EPT // CH
INIT_CHUNKS = ROWS_PT // CH
DEG_EPT = E // (NT * NC)
DEG_CH = 100
DEG_NCHUNK = DEG_EPT // DEG_CH

_MESH = plsc.VectorSubcoreMesh(core_axis_name="c", subcore_axis_name="s")
_SC_PARAMS = pltpu.CompilerParams(use_tc_tiling_on_sc=False,
                                  needs_layout_passes=False)


def _build_deg(interpret=False):
    return functools.partial(
        pl.kernel,
        out_type=jax.ShapeDtypeStruct((NC * N, 16), jnp.float32),
        mesh=_MESH,
        scratch_types=[
            pltpu.VMEM((DEG_NCHUNK, DEG_CH), jnp.int32),
            pltpu.VMEM((DEG_CH, 16), jnp.float32),
            pltpu.VMEM((ROWS_PT, 16), jnp.float32),
            pltpu.VMEM_SHARED((N, 16), jnp.float32),
            pltpu.SemaphoreType.DMA,
        ],
        compiler_params=_SC_PARAMS,
        interpret=interpret,
    )(_deg_body)


def _deg_body(dst_hbm, ones_hbm, out_hbm, dstbuf, ones_v, stage, acc, sem):
    c = lax.axis_index("c")
    s = lax.axis_index("s")
    w = c * NT + s
    pltpu.sync_copy(dst_hbm.at[w], dstbuf)
    pltpu.sync_copy(ones_hbm.at[pl.ds(0, DEG_CH)], ones_v)
    pltpu.sync_copy(ones_hbm, stage)
    pltpu.sync_copy(stage, acc.at[pl.ds(s * ROWS_PT, ROWS_PT)])
    plsc.subcore_barrier()

    def chunk(j, carry):
        pltpu.sync_copy(ones_v, acc.at[dstbuf.at[j]], add=True)
        return carry

    lax.fori_loop(0, DEG_NCHUNK, chunk, 0)
    plsc.subcore_barrier()
    pltpu.sync_copy(acc.at[pl.ds(s * ROWS_PT, ROWS_PT)], stage)
    pltpu.sync_copy(stage, out_hbm.at[pl.ds(c * N + s * ROWS_PT, ROWS_PT)])


_deg_kernel = _build_deg()


def _make_prop(dh, interpret=False):
    @functools.partial(
        pl.kernel,
        out_type=jax.ShapeDtypeStruct((NC * N, dh), jnp.float32),
        mesh=_MESH,
        scratch_types=[
            pltpu.VMEM((NCHUNK, CH), jnp.int32),
            pltpu.VMEM((NCHUNK, CH), jnp.int32),
            pltpu.VMEM((CH, dh), jnp.float32),
            pltpu.VMEM_SHARED((N, dh), jnp.float32),
            pltpu.SemaphoreType.DMA,
        ],
        compiler_params=_SC_PARAMS,
        interpret=interpret,
    )
    def prop(y_hbm, srcb_hbm, dstb_hbm, out_hbm, srcb, dstb, rows, acc, sem):
        c = lax.axis_index("c")
        s = lax.axis_index("s")
        pltpu.sync_copy(srcb_hbm.at[c, s], srcb)
        pltpu.sync_copy(dstb_hbm.at[s], dstb)

        def init(k, carry):
            pltpu.sync_copy(y_hbm.at[pl.ds(c * N + s * ROWS_PT + k * CH, CH)],
                            rows)
            pltpu.sync_copy(rows, acc.at[pl.ds(s * ROWS_PT + k * CH, CH)])
            return carry

        lax.fori_loop(0, INIT_CHUNKS, init, 0)
        plsc.subcore_barrier()

        def chunk(j, carry):
            pltpu.async_copy(y_hbm.at[srcb.at[j]], rows, sem).wait()
            pltpu.sync_copy(rows, acc.at[dstb.at[j]], add=True)
            return carry

        lax.fori_loop(0, NCHUNK, chunk, 0)
        plsc.subcore_barrier()

        def drain(k, carry):
            pltpu.sync_copy(acc.at[pl.ds(s * ROWS_PT + k * CH, CH)], rows)
            pltpu.sync_copy(rows,
                            out_hbm.at[pl.ds(c * N + s * ROWS_PT + k * CH, CH)])
            return carry

        lax.fori_loop(0, INIT_CHUNKS, drain, 0)

    return prop


_prop128 = _make_prop(128)
_prop64 = _make_prop(64)


_R = 1000
_GRID = N // _R


def _tc1_body(x_ref, w_ref, b_ref, d_ref, y_ref, dis_ref):
    dis = lax.rsqrt(d_ref[0][:, :1] + d_ref[1][:, :1] - 1.0)
    h = jnp.dot(x_ref[...], w_ref[...], preferred_element_type=jnp.float32)
    h = (h + b_ref[...]) * dis
    y_ref[0] = h[:, :128]
    y_ref[1] = h[:, 128:]
    dis_ref[...] = dis


def _tc1(x, w1, b1, deg):
    return pl.pallas_call(
        _tc1_body,
        grid=(_GRID,),
        in_specs=[
            pl.BlockSpec((_R, IN_DIM), lambda i: (i, 0)),
            pl.BlockSpec((IN_DIM, HID_DIM), lambda i: (0, 0)),
            pl.BlockSpec((1, HID_DIM), lambda i: (0, 0)),
            pl.BlockSpec((NC, _R, 16), lambda i: (0, i, 0)),
        ],
        out_specs=[
            pl.BlockSpec((NC, _R, 128), lambda i: (0, i, 0)),
            pl.BlockSpec((_R, 1), lambda i: (i, 0)),
        ],
        out_shape=[
            jax.ShapeDtypeStruct((NC, N, 128), jnp.float32),
            jax.ShapeDtypeStruct((N, 1), jnp.float32),
        ],
    )(x, w1, b1, deg)


def _tc2_body(a_ref, w_ref, b_ref, dis_ref, y_ref):
    dis = dis_ref[...]
    h0 = jnp.maximum(a_ref[0] * dis, 0.0)
    h1 = jnp.maximum(a_ref[1] * dis, 0.0)
    y = jnp.dot(h0, w_ref[:128], preferred_element_type=jnp.float32)
    y = y + jnp.dot(h1, w_ref[128:], preferred_element_type=jnp.float32)
    y = (y + b_ref[...]) * dis
    y_ref[0] = y[:, :64]
    y_ref[1] = y[:, 64:]


def _tc2(acc1, w2, b2, dis):
    return pl.pallas_call(
        _tc2_body,
        grid=(_GRID,),
        in_specs=[
            pl.BlockSpec((NC, _R, 128), lambda i: (0, i, 0)),
            pl.BlockSpec((HID_DIM, LAT_DIM), lambda i: (0, 0)),
            pl.BlockSpec((1, LAT_DIM), lambda i: (0, 0)),
            pl.BlockSpec((_R, 1), lambda i: (i, 0)),
        ],
        out_specs=pl.BlockSpec((NC, _R, 64), lambda i: (0, i, 0)),
        out_shape=jax.ShapeDtypeStruct((NC, N, 64), jnp.float32),
    )(acc1, w2, b2, dis)


def _tc3_body(a_ref, w_ref, b_ref, dis_ref, y_ref):
    dis = dis_ref[...]
    z0 = a_ref[0] * dis
    z1 = a_ref[1] * dis
    y = jnp.dot(z0, w_ref[:64], preferred_element_type=jnp.float32)
    y = y + jnp.dot(z1, w_ref[64:], preferred_element_type=jnp.float32)
    y = (y + b_ref[...]) * dis
    y_ref[0] = y[:, :128]
    y_ref[1] = y[:, 128:]


def _tc3(acc2, w3, b3, dis):
    return pl.pallas_call(
        _tc3_body,
        grid=(_GRID,),
        in_specs=[
            pl.BlockSpec((NC, _R, 64), lambda i: (0, i, 0)),
            pl.BlockSpec((LAT_DIM, OUT_DIM), lambda i: (0, 0)),
            pl.BlockSpec((1, OUT_DIM), lambda i: (0, 0)),
            pl.BlockSpec((_R, 1), lambda i: (i, 0)),
        ],
        out_specs=pl.BlockSpec((NC, _R, 128), lambda i: (0, i, 0)),
        out_shape=jax.ShapeDtypeStruct((NC, N, 128), jnp.float32),
    )(acc2, w3, b3, dis)


def _tc4_body(a_ref, dis_ref, o_ref):
    dis = dis_ref[...]
    o_ref[:, :128] = a_ref[0] * dis
    o_ref[:, 128:] = a_ref[1] * dis


def _tc4(acc3, dis):
    return pl.pallas_call(
        _tc4_body,
        grid=(_GRID,),
        in_specs=[
            pl.BlockSpec((NC, _R, 128), lambda i: (0, i, 0)),
            pl.BlockSpec((_R, 1), lambda i: (i, 0)),
        ],
        out_specs=pl.BlockSpec((_R, OUT_DIM), lambda i: (i, 0)),
        out_shape=jax.ShapeDtypeStruct((N, OUT_DIM), jnp.float32),
    )(acc3, dis)


def kernel(x, edge_index, W1, b1, W2, b2, W3, b3):
    src = edge_index[0].astype(jnp.int32)
    dst = edge_index[1].astype(jnp.int32)
    dstb = dst.reshape(NT, NCHUNK, CH)
    srcb = jnp.stack([src, src + N]).reshape(NC, NT, NCHUNK, CH)
    dst_deg = dst.reshape(NC * NT, DEG_NCHUNK, DEG_CH)
    ones16 = jnp.ones((ROWS_PT, 16), jnp.float32)

    deg = _deg_kernel(dst_deg, ones16).reshape(NC, N, 16)

    y1, dis = _tc1(x, W1, b1.reshape(1, HID_DIM), deg)
    acc1 = _prop128(y1.reshape(NC * N, 128), srcb, dstb)
    y2 = _tc2(acc1.reshape(NC, N, 128), W2, b2.reshape(1, LAT_DIM), dis)
    acc2 = _prop64(y2.reshape(NC * N, 64), srcb, dstb)
    y3 = _tc3(acc2.reshape(NC, N, 64), W3, b3.reshape(1, OUT_DIM), dis)
    acc3 = _prop128(y3.reshape(NC * N, 128), srcb, dstb)
    return _tc4(acc3.reshape(NC, N, 128), dis)
